# baseline (speedup 1.0000x reference)
import functools

import jax
import jax.numpy as jnp
from jax import lax
from jax.experimental import pallas as pl
from jax.experimental.pallas import tpu as pltpu
from jax.experimental.pallas import tpu_sc as plsc

NUM_CORES = 2
NUM_SUBCORES = 16
NW = NUM_CORES * NUM_SUBCORES

VOCAB = 1_000_000
EMB = 32
SEQ = 50
BATCH = 16384
B_PER_W = BATCH // NW

SROWS = VOCAB // 4

STCH = 256
N_STCH = VOCAB // STCH
STAIL = VOCAB - N_STCH * STCH
SRING = 4

CHUNK = 128
N_CH2 = SEQ * (B_PER_W // CHUNK)
GBUF = 2


def _stage_body(table_hbm, tail_hbm, staged_hbm, tcs, sts, tl_v, lsems, ssems):
    cid = lax.axis_index("c")
    sid = lax.axis_index("s")
    wid = sid * NUM_CORES + cid
    lanes = lax.iota(jnp.int32, 16)
    diags = [lax.bitwise_and(lanes + t, 15) for t in range(16)]

    n_mine = (N_STCH - wid + NW - 1) // NW

    def start_load(c, j):
        pltpu.async_copy(table_hbm.at[:, pl.ds(c * STCH, STCH)], tcs[j],
                         lsems[j])

    def wait_load(j):
        pltpu.make_async_copy(table_hbm.at[:, pl.ds(0, STCH)], tcs[j],
                              lsems[j]).wait()

    def start_store(c, j):
        pltpu.async_copy(sts[j], staged_hbm.at[pl.ds(c * (STCH // 4),
                                                     STCH // 4)], ssems[j])

    def wait_store(j):
        pltpu.make_async_copy(sts[j], staged_hbm.at[pl.ds(0, STCH // 4)],
                              ssems[j]).wait()

    def transpose(src, dst, width):
        @pl.loop(0, width // 16, unroll=2)
        def _(ii):
            iv = ii * 16 + lanes
            qloc = lax.shift_right_logical(iv, 2)
            colbase = lax.shift_left(lax.bitwise_and(iv, 3), 5)
            for d0 in (0, 16):
                for t in range(16):
                    dvec = d0 + diags[t]
                    vec = plsc.load_gather(src, [dvec, iv])
                    plsc.store_scatter(dst, [qloc, colbase + dvec], vec)

    for j in range(SRING):
        start_load(wid + NW * j, j)

    @pl.loop(0, n_mine, step=SRING)
    def _run(t):
        for j in range(SRING):
            tj = t + j

            @pl.when(tj < n_mine)
            def _():
                wait_load(j)

                @pl.when(tj >= SRING)
                def _():
                    wait_store(j)

                transpose(tcs[j], sts[j], STCH)
                start_store(wid + NW * tj, j)

                @pl.when(tj + SRING < n_mine)
                def _():
                    start_load(wid + NW * (tj + SRING), j)

    for j in range(SRING):
        @pl.when(n_mine > j)
        def _():
            wait_store(j)

    @pl.when(wid == NW - 1)
    def _tail():
        pltpu.async_copy(tail_hbm, tl_v, lsems[0]).wait()
        transpose(tl_v, sts[0], STAIL)
        pltpu.async_copy(
            sts[0].at[pl.ds(0, STAIL // 4)],
            staged_hbm.at[pl.ds(N_STCH * (STCH // 4), STAIL // 4)],
            ssems[0]).wait()


@jax.jit
def _stage(table_t, tail_t):
    mesh = plsc.VectorSubcoreMesh(core_axis_name="c", subcore_axis_name="s")
    return pl.kernel(
        _stage_body,
        out_type=jax.ShapeDtypeStruct((SROWS, 128), jnp.float32),
        mesh=mesh,
        compiler_params=pltpu.CompilerParams(
            use_tc_tiling_on_sc=True, needs_layout_passes=False),
        scratch_types=(
            [[pltpu.VMEM((EMB, STCH), jnp.float32) for _ in range(SRING)]]
            + [[pltpu.VMEM((STCH // 4, 128), jnp.float32)
                for _ in range(SRING)]]
            + [pltpu.VMEM((EMB, STAIL), jnp.float32)]
            + [[pltpu.SemaphoreType.DMA for _ in range(SRING)]]
            + [[pltpu.SemaphoreType.DMA for _ in range(SRING)]]
        ),
    )(table_t, tail_t)


def _gather_body(idx_hbm, staged, out_hbm, idx_v, qv_list, gb_list,
                 tt_a, tt_b, sem_q, gsem_list, osem_a, osem_b):
    cid = lax.axis_index("c")
    sid = lax.axis_index("s")
    wid = sid * NUM_CORES + cid
    lanes = lax.iota(jnp.int32, 16)

    pltpu.async_copy(
        idx_hbm.at[:, pl.ds(wid * B_PER_W, B_PER_W)], idx_v, sem_q).wait()

    def build_q(m, slot):
        s = lax.shift_right_logical(m, 2)
        cc = lax.bitwise_and(m, 3)

        @pl.loop(0, CHUNK // 16)
        def _(kk):
            k0 = kk * 16
            r = idx_v[s, pl.ds(cc * CHUNK + k0, 16)]
            qv_list[slot][pl.ds(k0, 16)] = lax.shift_right_logical(r, 2)

    def start_gather(slot):
        pltpu.async_copy(staged.at[qv_list[slot]], gb_list[slot],
                         gsem_list[slot])

    def wait_gather(slot):
        pltpu.make_async_copy(staged.at[qv_list[slot]], gb_list[slot],
                              gsem_list[slot]).wait()

    diags = [lax.bitwise_and(lanes + t, 15) for t in range(16)]

    def extract(m, slot, tt_v):
        s = lax.shift_right_logical(m, 2)
        cc = lax.bitwise_and(m, 3)
        gb = gb_list[slot]

        @pl.loop(0, CHUNK // 16)
        def _(kk):
            k0 = kk * 16
            r = idx_v[s, pl.ds(cc * CHUNK + k0, 16)]
            col0 = lax.shift_left(lax.bitwise_and(r, 3), 5)
            kv = k0 + lanes
            for d0 in (0, 16):
                for t in range(16):
                    dvec = d0 + diags[t]
                    vec = plsc.load_gather(gb, [kv, col0 + dvec])
                    plsc.store_scatter(tt_v, [dvec, kv], vec)

    def out_ref(m):
        s = lax.shift_right_logical(m, 2)
        cc = lax.bitwise_and(m, 3)
        return out_hbm.at[s, :, pl.ds(wid * B_PER_W + cc * CHUNK, CHUNK)]

    def wait_store(tt_v, osem):
        pltpu.make_async_copy(
            tt_v, out_hbm.at[0, :, pl.ds(0, CHUNK)], osem).wait()

    for b in range(GBUF):
        build_q(jnp.int32(b), b)
        start_gather(b)

    tts = (tt_a, tt_b)
    osems = (osem_a, osem_b)

    @pl.loop(0, N_CH2, step=GBUF)
    def _p2(m):
        for j in range(GBUF):
            mj = m + j
            tb = j % 2

            if j >= 2:
                wait_store(tts[tb], osems[tb])
            else:
                @pl.when(m > 0)
                def _():
                    wait_store(tts[tb], osems[tb])

            wait_gather(j)
            extract(mj, j, tts[tb])
            nxt = mj + GBUF

            @pl.when(nxt < N_CH2)
            def _():
                build_q(nxt, j)
                start_gather(j)

            pltpu.async_copy(tts[tb], out_ref(mj), osems[tb])

    wait_store(tt_a, osem_a)
    wait_store(tt_b, osem_b)


@jax.jit
def _lookup(idx_t, staged):
    mesh = plsc.VectorSubcoreMesh(core_axis_name="c", subcore_axis_name="s")
    return pl.kernel(
        _gather_body,
        out_type=jax.ShapeDtypeStruct((SEQ, EMB, BATCH), jnp.float32),
        mesh=mesh,
        compiler_params=pltpu.CompilerParams(
            use_tc_tiling_on_sc=True, needs_layout_passes=False),
        scratch_types=(
            [pltpu.VMEM((SEQ, B_PER_W), jnp.int32)]
            + [[pltpu.VMEM((CHUNK,), jnp.int32) for _ in range(GBUF)]]
            + [[pltpu.VMEM((CHUNK, 128), jnp.float32) for _ in range(GBUF)]]
            + [pltpu.VMEM((EMB, CHUNK), jnp.float32) for _ in range(2)]
            + [pltpu.SemaphoreType.DMA]
            + [[pltpu.SemaphoreType.DMA for _ in range(GBUF)]]
            + [pltpu.SemaphoreType.DMA for _ in range(2)]
        ),
    )(idx_t, staged)


def kernel(idx, targets, embedding_table):
    del targets
    idx_t = idx.astype(jnp.int32).T
    table_t = embedding_table.T
    tail_t = embedding_table[N_STCH * STCH:].T
    staged = _stage(table_t, tail_t)
    out_t = _lookup(idx_t, staged)
    return jnp.transpose(out_t, (2, 0, 1))


# speedup vs baseline: 1.0123x; 1.0123x over previous
import functools

import jax
import jax.numpy as jnp
from jax import lax
from jax.experimental import pallas as pl
from jax.experimental.pallas import tpu as pltpu
from jax.experimental.pallas import tpu_sc as plsc

NUM_CORES = 2
NUM_SUBCORES = 16
NW = NUM_CORES * NUM_SUBCORES

VOCAB = 1_000_000
EMB = 32
SEQ = 50
BATCH = 16384
B_PER_W = BATCH // NW

SROWS = VOCAB // 4

STCH = 128
N_STCH = VOCAB // STCH
STAIL = VOCAB - N_STCH * STCH
SRING = 2

CHUNK = 128
N_CH2 = SEQ * (B_PER_W // CHUNK)
GBUF = 2


def _stage_body(table_hbm, tail_hbm, staged_hbm, tcs, sts, tl_v, lsems, ssems):
    cid = lax.axis_index("c")
    sid = lax.axis_index("s")
    wid = sid * NUM_CORES + cid
    lanes = lax.iota(jnp.int32, 16)
    diags = [lax.bitwise_and(lanes + t, 15) for t in range(16)]

    n_mine = (N_STCH - wid + NW - 1) // NW

    def start_load(c, j):
        pltpu.async_copy(table_hbm.at[:, pl.ds(c * STCH, STCH)], tcs[j],
                         lsems[j])

    def wait_load(j):
        pltpu.make_async_copy(table_hbm.at[:, pl.ds(0, STCH)], tcs[j],
                              lsems[j]).wait()

    def start_store(c, j):
        pltpu.async_copy(sts[j], staged_hbm.at[pl.ds(c * (STCH // 4),
                                                     STCH // 4)], ssems[j])

    def wait_store(j):
        pltpu.make_async_copy(sts[j], staged_hbm.at[pl.ds(0, STCH // 4)],
                              ssems[j]).wait()

    def transpose(src, dst, width):
        @pl.loop(0, width // 16)
        def _(ii):
            iv = ii * 16 + lanes
            qloc = lax.shift_right_logical(iv, 2)
            colbase = lax.shift_left(lax.bitwise_and(iv, 3), 5)
            for d0 in (0, 16):
                for t in range(16):
                    dvec = d0 + diags[t]
                    vec = plsc.load_gather(src, [dvec, iv])
                    plsc.store_scatter(dst, [qloc, colbase + dvec], vec)

    for j in range(SRING):
        start_load(wid + NW * j, j)

    @pl.loop(0, n_mine, step=SRING)
    def _run(t):
        for j in range(SRING):
            tj = t + j

            @pl.when(tj < n_mine)
            def _():
                wait_load(j)

                @pl.when(tj >= SRING)
                def _():
                    wait_store(j)

                transpose(tcs[j], sts[j], STCH)
                start_store(wid + NW * tj, j)

                @pl.when(tj + SRING < n_mine)
                def _():
                    start_load(wid + NW * (tj + SRING), j)

    for j in range(SRING):
        @pl.when(n_mine > j)
        def _():
            wait_store(j)

    @pl.when(wid == NW - 1)
    def _tail():
        pltpu.async_copy(tail_hbm, tl_v, lsems[0]).wait()
        transpose(tl_v, sts[0], STAIL)
        pltpu.async_copy(
            sts[0].at[pl.ds(0, STAIL // 4)],
            staged_hbm.at[pl.ds(N_STCH * (STCH // 4), STAIL // 4)],
            ssems[0]).wait()


@jax.jit
def _stage(table_t, tail_t):
    mesh = plsc.VectorSubcoreMesh(core_axis_name="c", subcore_axis_name="s")
    return pl.kernel(
        _stage_body,
        out_type=jax.ShapeDtypeStruct((SROWS, 128), jnp.float32),
        mesh=mesh,
        compiler_params=pltpu.CompilerParams(
            use_tc_tiling_on_sc=True, needs_layout_passes=False),
        scratch_types=(
            [[pltpu.VMEM((EMB, STCH), jnp.float32) for _ in range(SRING)]]
            + [[pltpu.VMEM((STCH // 4, 128), jnp.float32)
                for _ in range(SRING)]]
            + [pltpu.VMEM((EMB, STAIL), jnp.float32)]
            + [[pltpu.SemaphoreType.DMA for _ in range(SRING)]]
            + [[pltpu.SemaphoreType.DMA for _ in range(SRING)]]
        ),
    )(table_t, tail_t)


def _gather_body(idx_hbm, staged, out_hbm, idx_v, qv_list, gb_list,
                 tt_a, tt_b, sem_q, gsem_list, osem_a, osem_b):
    cid = lax.axis_index("c")
    sid = lax.axis_index("s")
    wid = sid * NUM_CORES + cid
    lanes = lax.iota(jnp.int32, 16)

    pltpu.async_copy(
        idx_hbm.at[:, pl.ds(wid * B_PER_W, B_PER_W)], idx_v, sem_q).wait()

    def build_q(m, slot):
        s = lax.shift_right_logical(m, 2)
        cc = lax.bitwise_and(m, 3)

        @pl.loop(0, CHUNK // 16)
        def _(kk):
            k0 = kk * 16
            r = idx_v[s, pl.ds(cc * CHUNK + k0, 16)]
            qv_list[slot][pl.ds(k0, 16)] = lax.shift_right_logical(r, 2)

    def start_gather(slot):
        pltpu.async_copy(staged.at[qv_list[slot]], gb_list[slot],
                         gsem_list[slot])

    def wait_gather(slot):
        pltpu.make_async_copy(staged.at[qv_list[slot]], gb_list[slot],
                              gsem_list[slot]).wait()

    diags = [lax.bitwise_and(lanes + t, 15) for t in range(16)]

    def extract(m, slot, tt_v):
        s = lax.shift_right_logical(m, 2)
        cc = lax.bitwise_and(m, 3)
        gb = gb_list[slot]

        @pl.loop(0, CHUNK // 16)
        def _(kk):
            k0 = kk * 16
            r = idx_v[s, pl.ds(cc * CHUNK + k0, 16)]
            col0 = lax.shift_left(lax.bitwise_and(r, 3), 5)
            kv = k0 + lanes
            for d0 in (0, 16):
                for t in range(16):
                    dvec = d0 + diags[t]
                    vec = plsc.load_gather(gb, [kv, col0 + dvec])
                    plsc.store_scatter(tt_v, [dvec, kv], vec)

    def out_ref(m):
        s = lax.shift_right_logical(m, 2)
        cc = lax.bitwise_and(m, 3)
        return out_hbm.at[s, :, pl.ds(wid * B_PER_W + cc * CHUNK, CHUNK)]

    def wait_store(tt_v, osem):
        pltpu.make_async_copy(
            tt_v, out_hbm.at[0, :, pl.ds(0, CHUNK)], osem).wait()

    for b in range(GBUF):
        build_q(jnp.int32(b), b)
        start_gather(b)

    tts = (tt_a, tt_b)
    osems = (osem_a, osem_b)

    @pl.loop(0, N_CH2, step=GBUF)
    def _p2(m):
        for j in range(GBUF):
            mj = m + j
            tb = j % 2

            if j >= 2:
                wait_store(tts[tb], osems[tb])
            else:
                @pl.when(m > 0)
                def _():
                    wait_store(tts[tb], osems[tb])

            wait_gather(j)
            extract(mj, j, tts[tb])
            nxt = mj + GBUF

            @pl.when(nxt < N_CH2)
            def _():
                build_q(nxt, j)
                start_gather(j)

            pltpu.async_copy(tts[tb], out_ref(mj), osems[tb])

    wait_store(tt_a, osem_a)
    wait_store(tt_b, osem_b)


@jax.jit
def _lookup(idx_t, staged):
    mesh = plsc.VectorSubcoreMesh(core_axis_name="c", subcore_axis_name="s")
    return pl.kernel(
        _gather_body,
        out_type=jax.ShapeDtypeStruct((SEQ, EMB, BATCH), jnp.float32),
        mesh=mesh,
        compiler_params=pltpu.CompilerParams(
            use_tc_tiling_on_sc=True, needs_layout_passes=False),
        scratch_types=(
            [pltpu.VMEM((SEQ, B_PER_W), jnp.int32)]
            + [[pltpu.VMEM((CHUNK,), jnp.int32) for _ in range(GBUF)]]
            + [[pltpu.VMEM((CHUNK, 128), jnp.float32) for _ in range(GBUF)]]
            + [pltpu.VMEM((EMB, CHUNK), jnp.float32) for _ in range(2)]
            + [pltpu.SemaphoreType.DMA]
            + [[pltpu.SemaphoreType.DMA for _ in range(GBUF)]]
            + [pltpu.SemaphoreType.DMA for _ in range(2)]
        ),
    )(idx_t, staged)


def kernel(idx, targets, embedding_table):
    del targets
    idx_t = idx.astype(jnp.int32).T
    table_t = embedding_table.T
    tail_t = embedding_table[N_STCH * STCH:].T
    staged = _stage(table_t, tail_t)
    out_t = _lookup(idx_t, staged)
    return jnp.transpose(out_t, (2, 0, 1))
